# Initial kernel scaffold; baseline (speedup 1.0000x reference)
import jax
import jax.numpy as jnp
from jax.experimental import pallas as pl


def kernel(x, W_patch, b_patch, cls_token, pos_embed, ln1_g, ln1_b, W_in, b_in, W_out, b_out, ln2_g, ln2_b, W_m1, b_m1, W_m2, b_m2, W_gate, b_gate, We1, be1, We2, be2, We3, be3, lnf_g, lnf_b, W_head, b_head):
    raise NotImplementedError("write your pallas kernel here")


# speedup vs baseline: 2.2204x; 2.2204x over previous
import functools
import math

import jax
import jax.numpy as jnp
from jax.experimental import pallas as pl
from jax.experimental.pallas import tpu as pltpu

B = 8
E = 384
P = 16
HP = 14
NTOK = 197
NPAD = 200
NH = 12
DH = 32
NEXP = 10
TOPK = 8
H1 = 1536
H2 = 3072
NCLS = 1000
NCLS_PAD = 1024
PD = 3 * P * P
H2BLK = 768
NB = H2 // H2BLK

_f32 = jnp.float32


def _gelu(v):
    return 0.5 * v * (1.0 + jax.lax.erf(v * (1.0 / math.sqrt(2.0))))


def _lnorm(h, g, b, eps=1e-5):
    mu = jnp.mean(h, axis=-1, keepdims=True)
    var = jnp.mean((h - mu) ** 2, axis=-1, keepdims=True)
    return (h - mu) / jnp.sqrt(var + eps) * g + b


def _embed_kv_body(p_ref, wp_ref, badd_ref, g_ref, b_ref, wkv_ref, bkv_ref,
                   h0_ref, kv_ref):
    pb = p_ref[0]
    h = jnp.dot(pb, wp_ref[...], preferred_element_type=_f32, precision=jax.lax.Precision.HIGHEST) + badd_ref[...]
    h0_ref[0] = h[0:1]
    hn = _lnorm(h, g_ref[...], b_ref[...])
    kv = jax.lax.dot_general(hn, wkv_ref[...], (((1,), (1,)), ((), ())),
                             preferred_element_type=_f32, precision=jax.lax.Precision.HIGHEST) + bkv_ref[...]
    kv_ref[0] = kv


def _attn_body(kv_ref, h0_ref, g_ref, b_ref, wq_ref, bq_ref, ao_ref):
    h0 = h0_ref[0]
    hn0 = _lnorm(h0, g_ref[...], b_ref[...])
    q0 = jax.lax.dot_general(hn0, wq_ref[...], (((1,), (1,)), ((), ())),
                             preferred_element_type=_f32, precision=jax.lax.Precision.HIGHEST) + bq_ref[...]
    kvb = kv_ref[0]
    kb = kvb[:, 0:E]
    vb = kvb[:, E:2 * E]
    ri = jax.lax.broadcasted_iota(jnp.int32, (E, 128), 0)
    ci = jax.lax.broadcasted_iota(jnp.int32, (E, 128), 1)
    gmat = ((ri // DH) == ci).astype(_f32)
    s = jnp.dot(kb * q0, gmat, preferred_element_type=_f32, precision=jax.lax.Precision.HIGHEST)
    s = s * (1.0 / math.sqrt(DH))
    rows = jax.lax.broadcasted_iota(jnp.int32, (NPAD, 128), 0)
    s = jnp.where(rows < NTOK, s, -1e30)
    mx = jnp.max(s, axis=0, keepdims=True)
    e = jnp.exp(s - mx)
    att = e / jnp.sum(e, axis=0, keepdims=True)
    ri2 = jax.lax.broadcasted_iota(jnp.int32, (128, E), 0)
    ci2 = jax.lax.broadcasted_iota(jnp.int32, (128, E), 1)
    gmat2 = ((ci2 // DH) == ri2).astype(_f32)
    a2 = jnp.dot(att, gmat2, preferred_element_type=_f32, precision=jax.lax.Precision.HIGHEST)
    ao_ref[0] = jnp.sum(vb * a2, axis=0, keepdims=True)


def _mlp_router_body(ao_ref, h0_ref, wo_ref, bo_ref, g2_ref, b2_ref,
                     wm1_ref, bm1_ref, wm2_ref, bm2_ref, wg_ref, bg_ref,
                     hout_ref, comb_ref):
    cd = (((1,), (1,)), ((), ()))
    h0 = h0_ref[...] + jax.lax.dot_general(
        ao_ref[...], wo_ref[...], cd, preferred_element_type=_f32, precision=jax.lax.Precision.HIGHEST) + bo_ref[...]
    m = _lnorm(h0, g2_ref[...], b2_ref[...])
    u = _gelu(jax.lax.dot_general(m, wm1_ref[...], cd,
                                  preferred_element_type=_f32, precision=jax.lax.Precision.HIGHEST) + bm1_ref[...])
    mm = jax.lax.dot_general(u, wm2_ref[...], cd,
                             preferred_element_type=_f32, precision=jax.lax.Precision.HIGHEST) + bm2_ref[...]
    h0f = h0 + mm
    hout_ref[...] = h0f
    s = jax.lax.dot_general(h0f, wg_ref[...], cd,
                            preferred_element_type=_f32, precision=jax.lax.Precision.HIGHEST) + bg_ref[...]
    lane = jax.lax.broadcasted_iota(jnp.int32, (B, 128), 1)
    rank = jnp.zeros((B, 128), jnp.int32)
    for j in range(NEXP):
        sj = s[:, j:j + 1]
        rank = rank + ((sj > s) | ((sj == s) & (j < lane))).astype(jnp.int32)
    keep = ((rank < TOPK) & (lane < NEXP)).astype(_f32)
    mx = jnp.max(s, axis=1, keepdims=True)
    e = jnp.exp(s - mx) * keep
    comb_ref[...] = e / jnp.sum(e, axis=1, keepdims=True)


def _expert_body(h_ref, w1_ref, b1_ref, w2_ref, b2_ref, w3_ref, b3_ref,
                 c_ref, out_ref, u1_s, acc_s):
    xg = pl.program_id(0)
    j = pl.program_id(1)
    cd = (((1,), (1,)), ((), ()))

    @pl.when(j == 0)
    def _():
        b1 = b1_ref[pl.ds(xg, 1), :]
        u1_s[...] = _gelu(jax.lax.dot_general(
            h_ref[...], w1_ref[0], cd, preferred_element_type=_f32, precision=jax.lax.Precision.HIGHEST) + b1)

    b2 = b2_ref[0, pl.ds(xg, 1), :]
    u2 = _gelu(jax.lax.dot_general(u1_s[...], w2_ref[0], cd,
                                   preferred_element_type=_f32, precision=jax.lax.Precision.HIGHEST) + b2)
    part = jax.lax.dot_general(u2, w3_ref[0], cd, preferred_element_type=_f32, precision=jax.lax.Precision.HIGHEST)

    @pl.when(j == 0)
    def _():
        acc_s[...] = part

    @pl.when(j > 0)
    def _():
        acc_s[...] = acc_s[...] + part

    @pl.when(j == NB - 1)
    def _():
        e3 = acc_s[...] + b3_ref[pl.ds(xg, 1), :]
        contrib = e3 * c_ref[0][:, 0:1]
        prev = jnp.where(xg == 0, jnp.zeros_like(contrib), out_ref[...])
        out_ref[...] = prev + contrib


def _head_body(h_ref, g_ref, b_ref, wh_ref, bh_ref, out_ref):
    hn = _lnorm(h_ref[...], g_ref[...], b_ref[...])
    out_ref[...] = jax.lax.dot_general(
        hn, wh_ref[...], (((1,), (1,)), ((), ())),
        preferred_element_type=_f32, precision=jax.lax.Precision.HIGHEST) + bh_ref[...]


def kernel(x, W_patch, b_patch, cls_token, pos_embed, ln1_g, ln1_b, W_in,
           b_in, W_out, b_out, ln2_g, ln2_b, W_m1, b_m1, W_m2, b_m2, W_gate,
           b_gate, We1, be1, We2, be2, We3, be3, lnf_g, lnf_b, W_head,
           b_head):
    patches = x.reshape(B, 3, HP, P, HP, P).transpose(0, 2, 4, 1, 3, 5)
    patches = patches.reshape(B, HP * HP, PD)
    ppad = jnp.pad(patches, ((0, 0), (1, NPAD - NTOK), (0, 0)))
    wp2 = W_patch.reshape(E, PD).T
    badd = jnp.concatenate(
        [cls_token[0] + pos_embed[0, :1], b_patch[None, :] + pos_embed[0, 1:]],
        axis=0)
    badd = jnp.pad(badd, ((0, NPAD - NTOK), (0, 0)))
    w_q = W_in[:E]
    b_q = b_in[:E].reshape(1, E)
    w_kv = W_in[E:3 * E]
    b_kv = b_in[E:3 * E].reshape(1, 2 * E)
    g1 = ln1_g.reshape(1, E)
    bb1 = ln1_b.reshape(1, E)
    wg_pad = jnp.pad(W_gate, ((0, 128 - NEXP), (0, 0)))
    bg_pad = jnp.pad(b_gate.reshape(1, NEXP), ((0, 0), (0, 128 - NEXP)),
                     constant_values=-1e30)
    be2r = be2.reshape(NEXP, NB, H2BLK).transpose(1, 0, 2)
    wh_pad = jnp.pad(W_head, ((0, NCLS_PAD - NCLS), (0, 0)))
    bh_pad = jnp.pad(b_head.reshape(1, NCLS), ((0, 0), (0, NCLS_PAD - NCLS)))

    full = lambda *shape: pl.BlockSpec(shape, lambda *_: tuple(0 for _ in shape))

    h0pre, kv = pl.pallas_call(
        _embed_kv_body,
        grid=(B,),
        in_specs=[
            pl.BlockSpec((1, NPAD, PD), lambda b: (b, 0, 0)),
            full(PD, E),
            full(NPAD, E),
            full(1, E),
            full(1, E),
            full(2 * E, E),
            full(1, 2 * E),
        ],
        out_specs=[
            pl.BlockSpec((1, 1, E), lambda b: (b, 0, 0)),
            pl.BlockSpec((1, NPAD, 2 * E), lambda b: (b, 0, 0)),
        ],
        out_shape=[
            jax.ShapeDtypeStruct((B, 1, E), _f32),
            jax.ShapeDtypeStruct((B, NPAD, 2 * E), _f32),
        ],
    )(ppad, wp2, badd, g1, bb1, w_kv, b_kv)

    ao = pl.pallas_call(
        _attn_body,
        grid=(B,),
        in_specs=[
            pl.BlockSpec((1, NPAD, 2 * E), lambda b: (b, 0, 0)),
            pl.BlockSpec((1, 1, E), lambda b: (b, 0, 0)),
            full(1, E),
            full(1, E),
            full(E, E),
            full(1, E),
        ],
        out_specs=pl.BlockSpec((1, 1, E), lambda b: (b, 0, 0)),
        out_shape=jax.ShapeDtypeStruct((B, 1, E), _f32),
    )(kv, h0pre, g1, bb1, w_q, b_q)

    ao2 = ao.reshape(B, E)
    h0pre2 = h0pre.reshape(B, E)

    h0f, comb = pl.pallas_call(
        _mlp_router_body,
        in_specs=[
            full(B, E), full(B, E),
            full(E, E), full(1, E),
            full(1, E), full(1, E),
            full(H1, E), full(1, H1),
            full(E, H1), full(1, E),
            full(128, E), full(1, 128),
        ],
        out_specs=[full(B, E), full(B, 128)],
        out_shape=[
            jax.ShapeDtypeStruct((B, E), _f32),
            jax.ShapeDtypeStruct((B, 128), _f32),
        ],
    )(ao2, h0pre2, W_out, b_out.reshape(1, E), ln2_g.reshape(1, E),
      ln2_b.reshape(1, E), W_m1, b_m1.reshape(1, H1), W_m2,
      b_m2.reshape(1, E), wg_pad, bg_pad)

    comb3 = jnp.broadcast_to(comb[:, :NEXP].T[:, :, None], (NEXP, B, 128))

    hmoe = pl.pallas_call(
        _expert_body,
        grid=(NEXP, NB),
        in_specs=[
            full(B, E),
            pl.BlockSpec((1, H1, E), lambda xg, j: (xg, 0, 0)),
            full(NEXP, H1),
            pl.BlockSpec((1, H2BLK, H1), lambda xg, j: (xg, j, 0)),
            pl.BlockSpec((1, NEXP, H2BLK), lambda xg, j: (j, 0, 0)),
            pl.BlockSpec((1, E, H2BLK), lambda xg, j: (xg, 0, j)),
            full(NEXP, E),
            pl.BlockSpec((1, B, 128), lambda xg, j: (xg, 0, 0)),
        ],
        out_specs=full(B, E),
        out_shape=jax.ShapeDtypeStruct((B, E), _f32),
        scratch_shapes=[
            pltpu.VMEM((B, H1), _f32),
            pltpu.VMEM((B, E), _f32),
        ],
    )(h0f, We1, be1, We2, be2r, We3, be3, comb3)

    logits = pl.pallas_call(
        _head_body,
        in_specs=[
            full(B, E), full(1, E), full(1, E),
            full(NCLS_PAD, E), full(1, NCLS_PAD),
        ],
        out_specs=full(B, NCLS_PAD),
        out_shape=jax.ShapeDtypeStruct((B, NCLS_PAD), _f32),
    )(hmoe, lnf_g.reshape(1, E), lnf_b.reshape(1, E), wh_pad, bh_pad)

    return logits[:, :NCLS]


# speedup vs baseline: 4.5460x; 2.0474x over previous
import functools
import math

import jax
import jax.numpy as jnp
from jax.experimental import pallas as pl
from jax.experimental.pallas import tpu as pltpu

B = 8
E = 384
P = 16
HP = 14
NTOK = 197
NPAD = 200
NH = 12
DH = 32
NEXP = 10
TOPK = 8
H1 = 1536
H2 = 3072
NCLS = 1000
NCLS_PAD = 1024
PD = 3 * P * P
H2BLK = 768
NB = H2 // H2BLK

_f32 = jnp.float32


def _gelu(v):
    return 0.5 * v * (1.0 + jax.lax.erf(v * (1.0 / math.sqrt(2.0))))


def _lnorm(h, g, b, eps=1e-5):
    mu = jnp.mean(h, axis=-1, keepdims=True)
    var = jnp.mean((h - mu) ** 2, axis=-1, keepdims=True)
    return (h - mu) / jnp.sqrt(var + eps) * g + b


def _embed_kv_body(p_ref, wp_ref, badd_ref, g_ref, b_ref, wkv_ref, bkv_ref,
                   h0_ref, kv_ref):
    xb = p_ref[0]
    pmat = jnp.transpose(xb, (1, 3, 0, 2, 4)).reshape(HP * HP, PD)
    pb = jnp.concatenate(
        [jnp.zeros((1, PD), _f32), pmat,
         jnp.zeros((NPAD - NTOK, PD), _f32)], axis=0)
    h = jnp.dot(pb, wp_ref[...], preferred_element_type=_f32, precision=jax.lax.Precision.HIGHEST) + badd_ref[...]
    h0_ref[0] = h[0:1]
    hn = _lnorm(h, g_ref[...], b_ref[...])
    kv = jax.lax.dot_general(hn, wkv_ref[...], (((1,), (1,)), ((), ())),
                             preferred_element_type=_f32, precision=jax.lax.Precision.HIGHEST) + bkv_ref[...]
    kv_ref[0] = kv


def _attn_body(kv_ref, h0_ref, g_ref, b_ref, wq_ref, bq_ref, ao_ref):
    h0 = h0_ref[0]
    hn0 = _lnorm(h0, g_ref[...], b_ref[...])
    q0 = jax.lax.dot_general(hn0, wq_ref[...], (((1,), (1,)), ((), ())),
                             preferred_element_type=_f32, precision=jax.lax.Precision.HIGHEST) + bq_ref[...]
    kvb = kv_ref[0]
    kb = kvb[:, 0:E]
    vb = kvb[:, E:2 * E]
    ri = jax.lax.broadcasted_iota(jnp.int32, (E, 128), 0)
    ci = jax.lax.broadcasted_iota(jnp.int32, (E, 128), 1)
    gmat = ((ri // DH) == ci).astype(_f32)
    s = jnp.dot(kb * q0, gmat, preferred_element_type=_f32, precision=jax.lax.Precision.HIGHEST)
    s = s * (1.0 / math.sqrt(DH))
    rows = jax.lax.broadcasted_iota(jnp.int32, (NPAD, 128), 0)
    s = jnp.where(rows < NTOK, s, -1e30)
    mx = jnp.max(s, axis=0, keepdims=True)
    e = jnp.exp(s - mx)
    att = e / jnp.sum(e, axis=0, keepdims=True)
    ri2 = jax.lax.broadcasted_iota(jnp.int32, (128, E), 0)
    ci2 = jax.lax.broadcasted_iota(jnp.int32, (128, E), 1)
    gmat2 = ((ci2 // DH) == ri2).astype(_f32)
    a2 = jnp.dot(att, gmat2, preferred_element_type=_f32, precision=jax.lax.Precision.HIGHEST)
    ao_ref[0] = jnp.sum(vb * a2, axis=0, keepdims=True)


def _mlp_router_body(ao_ref, h0_ref, wo_ref, bo_ref, g2_ref, b2_ref,
                     wm1_ref, bm1_ref, wm2_ref, bm2_ref, wg_ref, bg_ref,
                     hout_ref, comb_ref):
    cd = (((1,), (1,)), ((), ()))
    h0 = h0_ref[...] + jax.lax.dot_general(
        ao_ref[...], wo_ref[...], cd, preferred_element_type=_f32, precision=jax.lax.Precision.HIGHEST) + bo_ref[...]
    m = _lnorm(h0, g2_ref[...], b2_ref[...])
    u = _gelu(jax.lax.dot_general(m, wm1_ref[...], cd,
                                  preferred_element_type=_f32, precision=jax.lax.Precision.HIGHEST) + bm1_ref[...])
    mm = jax.lax.dot_general(u, wm2_ref[...], cd,
                             preferred_element_type=_f32, precision=jax.lax.Precision.HIGHEST) + bm2_ref[...]
    h0f = h0 + mm
    hout_ref[...] = h0f
    s = jax.lax.dot_general(h0f, wg_ref[...], cd,
                            preferred_element_type=_f32, precision=jax.lax.Precision.HIGHEST) + bg_ref[...]
    lane = jax.lax.broadcasted_iota(jnp.int32, (B, 128), 1)
    rank = jnp.zeros((B, 128), jnp.int32)
    for j in range(NEXP):
        sj = s[:, j:j + 1]
        rank = rank + ((sj > s) | ((sj == s) & (j < lane))).astype(jnp.int32)
    keep = ((rank < TOPK) & (lane < NEXP)).astype(_f32)
    mx = jnp.max(s, axis=1, keepdims=True)
    e = jnp.exp(s - mx) * keep
    comb_ref[...] = e / jnp.sum(e, axis=1, keepdims=True)


def _expert_body(h_ref, w1_ref, b1_ref, w2_ref, b2_ref, w3_ref, b3_ref,
                 c_ref, out_ref, u1_s, acc_s):
    xg = pl.program_id(0)
    j = pl.program_id(1)
    cd = (((1,), (1,)), ((), ()))

    @pl.when(j == 0)
    def _():
        b1 = b1_ref[pl.ds(xg, 1), :]
        u1_s[...] = _gelu(jax.lax.dot_general(
            h_ref[...].astype(jnp.bfloat16), w1_ref[0].astype(jnp.bfloat16),
            cd, preferred_element_type=_f32) + b1)

    b2 = b2_ref[0, pl.ds(xg, 1), :]
    u2 = _gelu(jax.lax.dot_general(
        u1_s[...].astype(jnp.bfloat16), w2_ref[0].astype(jnp.bfloat16), cd,
        preferred_element_type=_f32) + b2)
    part = jax.lax.dot_general(
        u2.astype(jnp.bfloat16), w3_ref[0].astype(jnp.bfloat16), cd,
        preferred_element_type=_f32)

    @pl.when(j == 0)
    def _():
        acc_s[...] = part

    @pl.when(j > 0)
    def _():
        acc_s[...] = acc_s[...] + part

    @pl.when(j == NB - 1)
    def _():
        e3 = acc_s[...] + b3_ref[pl.ds(xg, 1), :]
        contrib = e3 * c_ref[0][:, 0:1]
        prev = jnp.where(xg == 0, jnp.zeros_like(contrib), out_ref[...])
        out_ref[...] = prev + contrib


def _head_body(h_ref, g_ref, b_ref, wh_ref, bh_ref, out_ref):
    hn = _lnorm(h_ref[...], g_ref[...], b_ref[...])
    out_ref[...] = jax.lax.dot_general(
        hn, wh_ref[...], (((1,), (1,)), ((), ())),
        preferred_element_type=_f32, precision=jax.lax.Precision.HIGHEST) + bh_ref[...]


def kernel(x, W_patch, b_patch, cls_token, pos_embed, ln1_g, ln1_b, W_in,
           b_in, W_out, b_out, ln2_g, ln2_b, W_m1, b_m1, W_m2, b_m2, W_gate,
           b_gate, We1, be1, We2, be2, We3, be3, lnf_g, lnf_b, W_head,
           b_head):
    x5 = x.reshape(B, 3, HP, P, HP, P)
    wp2 = W_patch.reshape(E, PD).T
    badd = jnp.concatenate(
        [cls_token[0] + pos_embed[0, :1], b_patch[None, :] + pos_embed[0, 1:]],
        axis=0)
    badd = jnp.pad(badd, ((0, NPAD - NTOK), (0, 0)))
    w_q = W_in[:E]
    b_q = b_in[:E].reshape(1, E)
    w_kv = W_in[E:3 * E]
    b_kv = b_in[E:3 * E].reshape(1, 2 * E)
    g1 = ln1_g.reshape(1, E)
    bb1 = ln1_b.reshape(1, E)
    wg_pad = jnp.pad(W_gate, ((0, 128 - NEXP), (0, 0)))
    bg_pad = jnp.pad(b_gate.reshape(1, NEXP), ((0, 0), (0, 128 - NEXP)),
                     constant_values=-1e30)
    be2r = be2.reshape(NEXP, NB, H2BLK).transpose(1, 0, 2)
    wh_pad = jnp.pad(W_head, ((0, NCLS_PAD - NCLS), (0, 0)))
    bh_pad = jnp.pad(b_head.reshape(1, NCLS), ((0, 0), (0, NCLS_PAD - NCLS)))

    full = lambda *shape: pl.BlockSpec(shape, lambda *_: tuple(0 for _ in shape))

    h0pre, kv = pl.pallas_call(
        _embed_kv_body,
        grid=(B,),
        in_specs=[
            pl.BlockSpec((1, 3, HP, P, HP, P), lambda b: (b, 0, 0, 0, 0, 0)),
            full(PD, E),
            full(NPAD, E),
            full(1, E),
            full(1, E),
            full(2 * E, E),
            full(1, 2 * E),
        ],
        out_specs=[
            pl.BlockSpec((1, 1, E), lambda b: (b, 0, 0)),
            pl.BlockSpec((1, NPAD, 2 * E), lambda b: (b, 0, 0)),
        ],
        out_shape=[
            jax.ShapeDtypeStruct((B, 1, E), _f32),
            jax.ShapeDtypeStruct((B, NPAD, 2 * E), _f32),
        ],
    )(x5, wp2, badd, g1, bb1, w_kv, b_kv)

    ao = pl.pallas_call(
        _attn_body,
        grid=(B,),
        in_specs=[
            pl.BlockSpec((1, NPAD, 2 * E), lambda b: (b, 0, 0)),
            pl.BlockSpec((1, 1, E), lambda b: (b, 0, 0)),
            full(1, E),
            full(1, E),
            full(E, E),
            full(1, E),
        ],
        out_specs=pl.BlockSpec((1, 1, E), lambda b: (b, 0, 0)),
        out_shape=jax.ShapeDtypeStruct((B, 1, E), _f32),
    )(kv, h0pre, g1, bb1, w_q, b_q)

    ao2 = ao.reshape(B, E)
    h0pre2 = h0pre.reshape(B, E)

    h0f, comb = pl.pallas_call(
        _mlp_router_body,
        in_specs=[
            full(B, E), full(B, E),
            full(E, E), full(1, E),
            full(1, E), full(1, E),
            full(H1, E), full(1, H1),
            full(E, H1), full(1, E),
            full(128, E), full(1, 128),
        ],
        out_specs=[full(B, E), full(B, 128)],
        out_shape=[
            jax.ShapeDtypeStruct((B, E), _f32),
            jax.ShapeDtypeStruct((B, 128), _f32),
        ],
    )(ao2, h0pre2, W_out, b_out.reshape(1, E), ln2_g.reshape(1, E),
      ln2_b.reshape(1, E), W_m1, b_m1.reshape(1, H1), W_m2,
      b_m2.reshape(1, E), wg_pad, bg_pad)

    comb3 = jnp.broadcast_to(comb[:, :NEXP].T[:, :, None], (NEXP, B, 128))

    hmoe = pl.pallas_call(
        _expert_body,
        grid=(NEXP, NB),
        in_specs=[
            full(B, E),
            pl.BlockSpec((1, H1, E), lambda xg, j: (xg, 0, 0)),
            full(NEXP, H1),
            pl.BlockSpec((1, H2BLK, H1), lambda xg, j: (xg, j, 0)),
            pl.BlockSpec((1, NEXP, H2BLK), lambda xg, j: (j, 0, 0)),
            pl.BlockSpec((1, E, H2BLK), lambda xg, j: (xg, 0, j)),
            full(NEXP, E),
            pl.BlockSpec((1, B, 128), lambda xg, j: (xg, 0, 0)),
        ],
        out_specs=full(B, E),
        out_shape=jax.ShapeDtypeStruct((B, E), _f32),
        scratch_shapes=[
            pltpu.VMEM((B, H1), _f32),
            pltpu.VMEM((B, E), _f32),
        ],
    )(h0f, We1, be1, We2, be2r, We3, be3, comb3)

    logits = pl.pallas_call(
        _head_body,
        in_specs=[
            full(B, E), full(1, E), full(1, E),
            full(NCLS_PAD, E), full(1, NCLS_PAD),
        ],
        out_specs=full(B, NCLS_PAD),
        out_shape=jax.ShapeDtypeStruct((B, NCLS_PAD), _f32),
    )(hmoe, lnf_g.reshape(1, E), lnf_b.reshape(1, E), wh_pad, bh_pad)

    return logits[:, :NCLS]


# speedup vs baseline: 5.0490x; 1.1107x over previous
import math

import jax
import jax.numpy as jnp
from jax.experimental import pallas as pl
from jax.experimental.pallas import tpu as pltpu

B = 8
E = 384
P = 16
HP = 14
NTOK = 197
NPAD = 200
NH = 12
DH = 32
NEXP = 10
TOPK = 8
H1 = 1536
H2 = 3072
NCLS = 1000
NCLS_PAD = 1024
PD = 3 * P * P
H2BLK = 1536
NB = H2 // H2BLK

_f32 = jnp.float32
_HI = jax.lax.Precision.HIGHEST


def _gelu(v):
    return 0.5 * v * (1.0 + jax.lax.erf(v * (1.0 / math.sqrt(2.0))))


def _lnorm(h, g, b, eps=1e-5):
    mu = jnp.mean(h, axis=-1, keepdims=True)
    var = jnp.mean((h - mu) ** 2, axis=-1, keepdims=True)
    return (h - mu) / jnp.sqrt(var + eps) * g + b


def _vit_body(x_ref, wp_ref, badd_ref, g1_ref, b1_ref, wkv_ref, bkv_ref,
              wq_ref, bq_ref, wo_ref, bo_ref, g2_ref, b2_ref, wm1_ref,
              bm1_ref, wm2_ref, bm2_ref, wg_ref, bg_ref,
              hout_ref, comb_ref, hpre_s, ao_s):
    b = pl.program_id(0)
    cd = (((1,), (1,)), ((), ()))

    xb = x_ref[0]
    pmat = jnp.transpose(xb, (1, 3, 0, 2, 4)).reshape(HP * HP, PD)
    pb = jnp.concatenate(
        [jnp.zeros((1, PD), _f32), pmat,
         jnp.zeros((NPAD - NTOK, PD), _f32)], axis=0)
    h = jnp.dot(pb, wp_ref[...], preferred_element_type=_f32,
                precision=_HI) + badd_ref[...]
    hpre_s[pl.ds(b, 1), :] = h[0:1]

    hn = _lnorm(h, g1_ref[...], b1_ref[...])
    kv = jax.lax.dot_general(hn, wkv_ref[...], cd,
                             preferred_element_type=_f32,
                             precision=_HI) + bkv_ref[...]
    q0 = jax.lax.dot_general(hn[0:1], wq_ref[...], cd,
                             preferred_element_type=_f32,
                             precision=_HI) + bq_ref[...]

    kb = kv[:, 0:E]
    vb = kv[:, E:2 * E]
    ri = jax.lax.broadcasted_iota(jnp.int32, (E, 128), 0)
    ci = jax.lax.broadcasted_iota(jnp.int32, (E, 128), 1)
    gmat = ((ri // DH) == ci).astype(_f32)
    s = jnp.dot(kb * q0, gmat, preferred_element_type=_f32, precision=_HI)
    s = s * (1.0 / math.sqrt(DH))
    rows = jax.lax.broadcasted_iota(jnp.int32, (NPAD, 128), 0)
    s = jnp.where(rows < NTOK, s, -1e30)
    mx = jnp.max(s, axis=0, keepdims=True)
    e = jnp.exp(s - mx)
    att = e / jnp.sum(e, axis=0, keepdims=True)
    ri2 = jax.lax.broadcasted_iota(jnp.int32, (128, E), 0)
    ci2 = jax.lax.broadcasted_iota(jnp.int32, (128, E), 1)
    gmat2 = ((ci2 // DH) == ri2).astype(_f32)
    a2 = jnp.dot(att, gmat2, preferred_element_type=_f32, precision=_HI)
    ao_s[pl.ds(b, 1), :] = jnp.sum(vb * a2, axis=0, keepdims=True)

    @pl.when(b == B - 1)
    def _():
        h0 = hpre_s[...] + jax.lax.dot_general(
            ao_s[...], wo_ref[...], cd, preferred_element_type=_f32,
            precision=_HI) + bo_ref[...]
        m = _lnorm(h0, g2_ref[...], b2_ref[...])
        u = _gelu(jax.lax.dot_general(m, wm1_ref[...], cd,
                                      preferred_element_type=_f32,
                                      precision=_HI) + bm1_ref[...])
        mm = jax.lax.dot_general(u, wm2_ref[...], cd,
                                 preferred_element_type=_f32,
                                 precision=_HI) + bm2_ref[...]
        h0f = h0 + mm
        hout_ref[...] = h0f
        sg = jax.lax.dot_general(h0f, wg_ref[...], cd,
                                 preferred_element_type=_f32,
                                 precision=_HI) + bg_ref[...]
        lane = jax.lax.broadcasted_iota(jnp.int32, (B, 128), 1)
        rank = jnp.zeros((B, 128), jnp.int32)
        for j in range(NEXP):
            sj = sg[:, j:j + 1]
            rank = rank + ((sj > sg) |
                           ((sj == sg) & (j < lane))).astype(jnp.int32)
        keep = ((rank < TOPK) & (lane < NEXP)).astype(_f32)
        mxg = jnp.max(sg, axis=1, keepdims=True)
        eg = jnp.exp(sg - mxg) * keep
        comb_ref[...] = eg / jnp.sum(eg, axis=1, keepdims=True)


def _expert_body(h_ref, w1_ref, b1_ref, w2_ref, b2_ref, w3_ref, b3_ref,
                 c_ref, gf_ref, bf_ref, wh_ref, bh_ref,
                 out_ref, u1_s, acc_s, moe_s):
    xg = pl.program_id(0)
    j = pl.program_id(1)
    cd = (((1,), (1,)), ((), ()))

    @pl.when(j == 0)
    def _():
        b1 = b1_ref[pl.ds(xg, 1), :]
        u1_s[...] = _gelu(jax.lax.dot_general(
            h_ref[...].astype(jnp.bfloat16), w1_ref[0].astype(jnp.bfloat16),
            cd, preferred_element_type=_f32) + b1)

    b2 = b2_ref[0, pl.ds(xg, 1), :]
    u2 = _gelu(jax.lax.dot_general(
        u1_s[...].astype(jnp.bfloat16), w2_ref[0].astype(jnp.bfloat16), cd,
        preferred_element_type=_f32) + b2)
    part = jax.lax.dot_general(
        u2.astype(jnp.bfloat16), w3_ref[0].astype(jnp.bfloat16), cd,
        preferred_element_type=_f32)

    @pl.when(j == 0)
    def _():
        acc_s[...] = part

    @pl.when(j > 0)
    def _():
        acc_s[...] = acc_s[...] + part

    @pl.when(j == NB - 1)
    def _():
        e3 = acc_s[...] + b3_ref[pl.ds(xg, 1), :]
        lane = jax.lax.broadcasted_iota(jnp.int32, (B, 128), 1)
        cw = jnp.sum(jnp.where(lane == xg, c_ref[...], 0.0),
                     axis=1, keepdims=True)
        contrib = e3 * cw
        prev = jnp.where(xg == 0, jnp.zeros_like(contrib), moe_s[...])
        moe_s[...] = prev + contrib

    @pl.when(jnp.logical_and(xg == NEXP - 1, j == NB - 1))
    def _():
        hn = _lnorm(moe_s[...], gf_ref[...], bf_ref[...])
        out_ref[...] = jax.lax.dot_general(
            hn, wh_ref[...], cd, preferred_element_type=_f32,
            precision=_HI) + bh_ref[...]


def kernel(x, W_patch, b_patch, cls_token, pos_embed, ln1_g, ln1_b, W_in,
           b_in, W_out, b_out, ln2_g, ln2_b, W_m1, b_m1, W_m2, b_m2, W_gate,
           b_gate, We1, be1, We2, be2, We3, be3, lnf_g, lnf_b, W_head,
           b_head):
    x5 = x.reshape(B, 3, HP, P, HP, P)
    wp2 = W_patch.reshape(E, PD).T
    badd = jnp.concatenate(
        [cls_token[0] + pos_embed[0, :1], b_patch[None, :] + pos_embed[0, 1:]],
        axis=0)
    badd = jnp.pad(badd, ((0, NPAD - NTOK), (0, 0)))
    w_q = W_in[:E]
    b_q = b_in[:E].reshape(1, E)
    w_kv = W_in[E:3 * E]
    b_kv = b_in[E:3 * E].reshape(1, 2 * E)
    wg_pad = jnp.pad(W_gate, ((0, 128 - NEXP), (0, 0)))
    bg_pad = jnp.pad(b_gate.reshape(1, NEXP), ((0, 0), (0, 128 - NEXP)),
                     constant_values=-1e30)
    be2r = be2.reshape(NEXP, NB, H2BLK).transpose(1, 0, 2)
    wh_pad = jnp.pad(W_head, ((0, NCLS_PAD - NCLS), (0, 0)))
    bh_pad = jnp.pad(b_head.reshape(1, NCLS), ((0, 0), (0, NCLS_PAD - NCLS)))

    full = lambda *shape: pl.BlockSpec(shape, lambda *_: tuple(0 for _ in shape))

    h0f, comb = pl.pallas_call(
        _vit_body,
        grid=(B,),
        in_specs=[
            pl.BlockSpec((1, 3, HP, P, HP, P),
                         lambda b: (b, 0, 0, 0, 0, 0)),
            full(PD, E),
            full(NPAD, E),
            full(1, E), full(1, E),
            full(2 * E, E), full(1, 2 * E),
            full(E, E), full(1, E),
            full(E, E), full(1, E),
            full(1, E), full(1, E),
            full(H1, E), full(1, H1),
            full(E, H1), full(1, E),
            full(128, E), full(1, 128),
        ],
        out_specs=[full(B, E), full(B, 128)],
        out_shape=[
            jax.ShapeDtypeStruct((B, E), _f32),
            jax.ShapeDtypeStruct((B, 128), _f32),
        ],
        scratch_shapes=[
            pltpu.VMEM((B, E), _f32),
            pltpu.VMEM((B, E), _f32),
        ],
    )(x5, wp2, badd, ln1_g.reshape(1, E), ln1_b.reshape(1, E), w_kv, b_kv,
      w_q, b_q, W_out, b_out.reshape(1, E), ln2_g.reshape(1, E),
      ln2_b.reshape(1, E), W_m1, b_m1.reshape(1, H1), W_m2,
      b_m2.reshape(1, E), wg_pad, bg_pad)

    logits = pl.pallas_call(
        _expert_body,
        grid=(NEXP, NB),
        in_specs=[
            full(B, E),
            pl.BlockSpec((1, H1, E), lambda xg, j: (xg, 0, 0)),
            full(NEXP, H1),
            pl.BlockSpec((1, H2BLK, H1), lambda xg, j: (xg, j, 0)),
            pl.BlockSpec((1, NEXP, H2BLK), lambda xg, j: (j, 0, 0)),
            pl.BlockSpec((1, E, H2BLK), lambda xg, j: (xg, 0, j)),
            full(NEXP, E),
            full(B, 128),
            full(1, E), full(1, E),
            full(NCLS_PAD, E), full(1, NCLS_PAD),
        ],
        out_specs=full(B, NCLS_PAD),
        out_shape=jax.ShapeDtypeStruct((B, NCLS_PAD), _f32),
        scratch_shapes=[
            pltpu.VMEM((B, H1), _f32),
            pltpu.VMEM((B, E), _f32),
            pltpu.VMEM((B, E), _f32),
        ],
    )(h0f, We1, be1, We2, be2r, We3, be3, comb,
      lnf_g.reshape(1, E), lnf_b.reshape(1, E), wh_pad, bh_pad)

    return logits[:, :NCLS]
